# baseline (speedup 1.0000x reference)
import jax
import jax.numpy as jnp
from jax.experimental import pallas as pl
from jax.experimental.pallas import tpu as pltpu

_NUM_EXPERTS = 16
_FREQ_BINS = 2097152
_ROWS = 8
_SUB = 262144
_NSUB = _FREQ_BINS // _SUB
_SLOTS = 4
_CHUNK = 8192
_NB = _NUM_EXPERTS - 1


def _mask_kernel(params_ref, out_ref, sb_ref, eb_ref, buf_ref, sem_ref):
    r = pl.program_id(0)
    t = (jax.nn.sigmoid(params_ref[:, :]) * (_FREQ_BINS - 1)).astype(jnp.int32)
    ts = [t[0, k] for k in range(_NB)]

    sb_ref[0] = 0
    for k in range(_NB):
        rank = jnp.int32(0)
        for m in range(_NB):
            if m == k:
                continue
            hit = (ts[m] <= ts[k]) if m < k else (ts[m] < ts[k])
            rank = rank + jnp.where(hit, 1, 0)
        sb_ref[rank + 1] = ts[k]
    for i in range(_NUM_EXPERTS - 1):
        eb_ref[i] = sb_ref[i + 1]
    eb_ref[_NUM_EXPERTS - 1] = _FREQ_BINS

    siota = jax.lax.broadcasted_iota(jnp.int32, (_ROWS, 1), 0) + r * _ROWS
    start_col = jnp.zeros((_ROWS, 1), jnp.int32)
    span_col = jnp.zeros((_ROWS, 1), jnp.int32)
    for i in range(_NUM_EXPERTS):
        start_col = jnp.where(siota == i, sb_ref[i], start_col)
        span_col = jnp.where(siota == i, eb_ref[i] - sb_ref[i], span_col)

    iota = jax.lax.broadcasted_iota(jnp.int32, (_ROWS, _CHUNK), 1)
    q0 = iota - jnp.broadcast_to(start_col, (_ROWS, _CHUNK))
    span_bc = jnp.broadcast_to(span_col, (_ROWS, _CHUNK)).astype(jnp.uint32)

    def _copy(slot, s):
        return pltpu.make_async_copy(
            buf_ref.at[slot],
            out_ref.at[pl.ds(r * _ROWS, _ROWS), s * _SUB:(s + 1) * _SUB],
            sem_ref.at[slot],
        )

    for s in range(_NSUB):
        slot = s % _SLOTS
        if s >= _SLOTS:
            _copy(slot, s - _SLOTS).wait()
        for c in range(_SUB // _CHUNK):
            q = (q0 + (s * _SUB + c * _CHUNK)).astype(jnp.uint32)
            buf_ref[slot, :, c * _CHUNK:(c + 1) * _CHUNK] = jnp.where(
                q < span_bc, 1.0, 0.0
            )
        _copy(slot, s).start()

    for s in range(_NSUB - _SLOTS, _NSUB):
        _copy(s % _SLOTS, s).wait()


def kernel(bound_params):
    params2d = bound_params.reshape(1, _NB)
    return pl.pallas_call(
        _mask_kernel,
        grid=(2,),
        in_specs=[
            pl.BlockSpec((1, _NB), lambda r: (0, 0)),
        ],
        out_specs=pl.BlockSpec(memory_space=pl.ANY),
        out_shape=jax.ShapeDtypeStruct((_NUM_EXPERTS, _FREQ_BINS), jnp.float32),
        scratch_shapes=[
            pltpu.SMEM((_NUM_EXPERTS,), jnp.int32),
            pltpu.SMEM((_NUM_EXPERTS,), jnp.int32),
            pltpu.VMEM((_SLOTS, _ROWS, _SUB), jnp.float32),
            pltpu.SemaphoreType.DMA((_SLOTS,)),
        ],
        compiler_params=pltpu.CompilerParams(
            dimension_semantics=("parallel",),
        ),
    )(params2d)


# speedup vs baseline: 1.1096x; 1.1096x over previous
import jax
import jax.numpy as jnp
from jax.experimental import pallas as pl
from jax.experimental.pallas import tpu as pltpu

_NUM_EXPERTS = 16
_FREQ_BINS = 2097152
_ROWS = 8
_BLOCK = 262144
_CHUNK = 8192
_GRID = (2, _FREQ_BINS // _BLOCK)
_NB = _NUM_EXPERTS - 1


def _mask_kernel(params_ref, out_ref, sb_ref, eb_ref):
    r = pl.program_id(0)
    g = pl.program_id(1)
    t = (jax.nn.sigmoid(params_ref[:, :]) * (_FREQ_BINS - 1)).astype(jnp.int32)
    ts = [t[0, k] for k in range(_NB)]

    sb_ref[0] = 0
    for k in range(_NB):
        rank = jnp.int32(0)
        for m in range(_NB):
            if m == k:
                continue
            hit = (ts[m] <= ts[k]) if m < k else (ts[m] < ts[k])
            rank = rank + jnp.where(hit, 1, 0)
        sb_ref[rank + 1] = ts[k]
    for i in range(_NUM_EXPERTS - 1):
        eb_ref[i] = sb_ref[i + 1]
    eb_ref[_NUM_EXPERTS - 1] = _FREQ_BINS

    siota = jax.lax.broadcasted_iota(jnp.int32, (_ROWS, 1), 0) + r * _ROWS
    start_col = jnp.zeros((_ROWS, 1), jnp.int32)
    span_col = jnp.zeros((_ROWS, 1), jnp.int32)
    for i in range(_NUM_EXPERTS):
        start_col = jnp.where(siota == i, sb_ref[i], start_col)
        span_col = jnp.where(siota == i, eb_ref[i] - sb_ref[i], span_col)

    iota = jax.lax.broadcasted_iota(jnp.int32, (_ROWS, _CHUNK), 1)
    q0 = iota - jnp.broadcast_to(start_col, (_ROWS, _CHUNK))
    span_bc = jnp.broadcast_to(span_col, (_ROWS, _CHUNK)).astype(jnp.uint32)
    for c in range(_BLOCK // _CHUNK):
        q = (q0 + (g * _BLOCK + c * _CHUNK)).astype(jnp.uint32)
        out_ref[:, c * _CHUNK:(c + 1) * _CHUNK] = jnp.where(
            q < span_bc, 1.0, 0.0
        )


def kernel(bound_params):
    params2d = bound_params.reshape(1, _NB)
    return pl.pallas_call(
        _mask_kernel,
        grid=_GRID,
        in_specs=[
            pl.BlockSpec((1, _NB), lambda r, g: (0, 0)),
        ],
        out_specs=pl.BlockSpec((_ROWS, _BLOCK), lambda r, g: (r, g)),
        out_shape=jax.ShapeDtypeStruct((_NUM_EXPERTS, _FREQ_BINS), jnp.float32),
        scratch_shapes=[
            pltpu.SMEM((_NUM_EXPERTS,), jnp.int32),
            pltpu.SMEM((_NUM_EXPERTS,), jnp.int32),
        ],
        compiler_params=pltpu.CompilerParams(
            dimension_semantics=("parallel", "arbitrary"),
        ),
    )(params2d)
